# baseline (speedup 1.0000x reference)
import functools

import jax
import jax.numpy as jnp
from jax.experimental import pallas as pl
from jax.experimental.pallas import tpu as pltpu


def _row_tile(n: int, target: int) -> int:
    best = n
    for t in range(8, target + 1, 8):
        if n % t == 0:
            best = t
    return best if best <= target or best == n else n


def _prologue_kernel(x_ref, W1_ref, linW_ref, linb_ref, Wg_ref, s1_ref, g_ref):
    xb = x_ref[...].astype(jnp.bfloat16)
    s1 = jnp.dot(xb, W1_ref[...].astype(jnp.bfloat16),
                 preferred_element_type=jnp.float32)
    s1_ref[...] = s1.astype(jnp.bfloat16)
    logits = jnp.dot(xb, linW_ref[...].astype(jnp.bfloat16),
                     preferred_element_type=jnp.float32) + linb_ref[...]
    wave = jax.nn.softmax(logits, axis=-1)
    g = jnp.dot(wave.astype(jnp.bfloat16), Wg_ref[...].astype(jnp.bfloat16),
                preferred_element_type=jnp.float32)
    g_ref[...] = g.astype(jnp.bfloat16)


def _gc_kernel(adj_ref, s1_ref, W2_ref, s2_ref):
    adjb = adj_ref[...].astype(jnp.bfloat16)
    h = jnp.dot(adjb, s1_ref[...], preferred_element_type=jnp.float32)
    h = jnp.maximum(h, 0.0)
    s2 = jnp.dot(h.astype(jnp.bfloat16), W2_ref[...].astype(jnp.bfloat16),
                 preferred_element_type=jnp.float32)
    s2_ref[...] = s2.astype(jnp.bfloat16)


def _emb_kernel(a_ref, adj_ref, k_ref, q_ref, s2_ref, g_ref, out_ref):
    a = a_ref[0]
    adjb = adj_ref[...].astype(jnp.bfloat16)
    acc1 = jnp.dot(adjb, s2_ref[...], preferred_element_type=jnp.float32)
    m = (a * k_ref[...] + (1.0 - a) * q_ref[...]).astype(jnp.bfloat16)
    acc2 = jnp.dot(m, g_ref[...], preferred_element_type=jnp.float32)
    out_ref[...] = jax.nn.softmax(acc1, axis=-1) + acc2


@functools.partial(jax.jit, static_argnames=())
def kernel(x, adj, q, k, W1, W2, lin_W, lin_b, Wg, apha):
    n, nfeat = x.shape
    nhid = W1.shape[1]
    nclass = W2.shape[1]

    a_sig = jax.nn.sigmoid(apha).reshape((1,))
    lin_b2 = lin_b.reshape((1, nclass))

    r0 = _row_tile(n, 1000)
    s1, g = pl.pallas_call(
        _prologue_kernel,
        grid=(n // r0,),
        in_specs=[
            pl.BlockSpec((r0, nfeat), lambda i: (i, 0)),
            pl.BlockSpec((nfeat, nhid), lambda i: (0, 0)),
            pl.BlockSpec((nfeat, nclass), lambda i: (0, 0)),
            pl.BlockSpec((1, nclass), lambda i: (0, 0)),
            pl.BlockSpec((nclass, nclass), lambda i: (0, 0)),
        ],
        out_specs=[
            pl.BlockSpec((r0, nhid), lambda i: (i, 0)),
            pl.BlockSpec((r0, nclass), lambda i: (i, 0)),
        ],
        out_shape=[
            jax.ShapeDtypeStruct((n, nhid), jnp.bfloat16),
            jax.ShapeDtypeStruct((n, nclass), jnp.bfloat16),
        ],
    )(x, W1, lin_W, lin_b2, Wg)

    r1 = _row_tile(n, 400)
    s2 = pl.pallas_call(
        _gc_kernel,
        grid=(n // r1,),
        in_specs=[
            pl.BlockSpec((r1, n), lambda i: (i, 0)),
            pl.BlockSpec((n, nhid), lambda i: (0, 0)),
            pl.BlockSpec((nhid, nclass), lambda i: (0, 0)),
        ],
        out_specs=pl.BlockSpec((r1, nclass), lambda i: (i, 0)),
        out_shape=jax.ShapeDtypeStruct((n, nclass), jnp.bfloat16),
    )(adj, s1, W2)

    r2 = _row_tile(n, 200)
    emb = pl.pallas_call(
        _emb_kernel,
        grid=(n // r2,),
        in_specs=[
            pl.BlockSpec(memory_space=pltpu.SMEM),
            pl.BlockSpec((r2, n), lambda i: (i, 0)),
            pl.BlockSpec((r2, n), lambda i: (i, 0)),
            pl.BlockSpec((r2, n), lambda i: (i, 0)),
            pl.BlockSpec((n, nclass), lambda i: (0, 0)),
            pl.BlockSpec((n, nclass), lambda i: (0, 0)),
        ],
        out_specs=pl.BlockSpec((r2, nclass), lambda i: (i, 0)),
        out_shape=jax.ShapeDtypeStruct((n, nclass), jnp.float32),
    )(a_sig, adj, k, q, s2, g)
    return emb


# speedup vs baseline: 1.0260x; 1.0260x over previous
import functools

import jax
import jax.numpy as jnp
from jax.experimental import pallas as pl
from jax.experimental.pallas import tpu as pltpu


def _row_tile(n: int, target: int) -> int:
    best = n
    for t in range(8, target + 1, 8):
        if n % t == 0:
            best = t
    return best


def _gc_kernel(adj_ref, x_ref, W1_ref, W2_ref, linW_ref, linb_ref, Wg_ref,
               s2_ref, g_ref):
    @pl.when(pl.program_id(0) == 0)
    def _():
        logits = jnp.dot(x_ref[...], linW_ref[...].astype(jnp.bfloat16),
                         preferred_element_type=jnp.float32) + linb_ref[...]
        wave = jax.nn.softmax(logits, axis=-1)
        g = jnp.dot(wave.astype(jnp.bfloat16), Wg_ref[...].astype(jnp.bfloat16),
                    preferred_element_type=jnp.float32)
        g_ref[...] = g.astype(jnp.bfloat16)

    adjb = adj_ref[...].astype(jnp.bfloat16)
    h = jnp.dot(adjb, x_ref[...], preferred_element_type=jnp.float32)
    h = jnp.dot(h.astype(jnp.bfloat16), W1_ref[...].astype(jnp.bfloat16),
                preferred_element_type=jnp.float32)
    h = jnp.maximum(h, 0.0)
    s2 = jnp.dot(h.astype(jnp.bfloat16), W2_ref[...].astype(jnp.bfloat16),
                 preferred_element_type=jnp.float32)
    s2_ref[...] = s2.astype(jnp.bfloat16)


def _emb_kernel(a_ref, adj_ref, k_ref, q_ref, s2_ref, g_ref, out_ref):
    a = a_ref[0]
    adjb = adj_ref[...].astype(jnp.bfloat16)
    acc1 = jnp.dot(adjb, s2_ref[...], preferred_element_type=jnp.float32)
    qv = q_ref[...]
    m = (qv + a * (k_ref[...] - qv)).astype(jnp.bfloat16)
    acc2 = jnp.dot(m, g_ref[...], preferred_element_type=jnp.float32)
    out_ref[...] = jax.nn.softmax(acc1, axis=-1) + acc2


@functools.partial(jax.jit, static_argnames=())
def kernel(x, adj, q, k, W1, W2, lin_W, lin_b, Wg, apha):
    n, nfeat = x.shape
    nhid = W1.shape[1]
    nclass = W2.shape[1]

    a_sig = jax.nn.sigmoid(apha).reshape((1,))
    lin_b2 = lin_b.reshape((1, nclass))
    x_bf = x.astype(jnp.bfloat16)

    r1 = _row_tile(n, 400)
    s2, g = pl.pallas_call(
        _gc_kernel,
        grid=(n // r1,),
        in_specs=[
            pl.BlockSpec((r1, n), lambda i: (i, 0)),
            pl.BlockSpec((n, nfeat), lambda i: (0, 0)),
            pl.BlockSpec((nfeat, nhid), lambda i: (0, 0)),
            pl.BlockSpec((nhid, nclass), lambda i: (0, 0)),
            pl.BlockSpec((nfeat, nclass), lambda i: (0, 0)),
            pl.BlockSpec((1, nclass), lambda i: (0, 0)),
            pl.BlockSpec((nclass, nclass), lambda i: (0, 0)),
        ],
        out_specs=[
            pl.BlockSpec((r1, nclass), lambda i: (i, 0)),
            pl.BlockSpec((n, nclass), lambda i: (0, 0)),
        ],
        out_shape=[
            jax.ShapeDtypeStruct((n, nclass), jnp.bfloat16),
            jax.ShapeDtypeStruct((n, nclass), jnp.bfloat16),
        ],
    )(adj, x_bf, W1, W2, lin_W, lin_b2, Wg)

    r2 = _row_tile(n, 200)
    emb = pl.pallas_call(
        _emb_kernel,
        grid=(n // r2,),
        in_specs=[
            pl.BlockSpec(memory_space=pltpu.SMEM),
            pl.BlockSpec((r2, n), lambda i: (i, 0)),
            pl.BlockSpec((r2, n), lambda i: (i, 0)),
            pl.BlockSpec((r2, n), lambda i: (i, 0)),
            pl.BlockSpec((n, nclass), lambda i: (0, 0)),
            pl.BlockSpec((n, nclass), lambda i: (0, 0)),
        ],
        out_specs=pl.BlockSpec((r2, nclass), lambda i: (i, 0)),
        out_shape=jax.ShapeDtypeStruct((n, nclass), jnp.float32),
    )(a_sig, adj, k, q, s2, g)
    return emb
